# baseline (speedup 1.0000x reference)
import functools

import jax
import jax.numpy as jnp
from jax import lax
from jax.experimental import pallas as pl
from jax.experimental.pallas import tpu as pltpu
from jax.experimental.pallas import tpu_sc as plsc

Z = 32
B = 16384

_NC = 2
_NS = 16
_NW = _NC * _NS
_BPW = B // _NW


def _make_kernel():
    mesh = plsc.VectorSubcoreMesh(core_axis_name="c", subcore_axis_name="s")

    @functools.partial(
        pl.kernel,
        mesh=mesh,
        out_type=(
            jax.ShapeDtypeStruct((B, Z), jnp.float32),
            jax.ShapeDtypeStruct((B, Z), jnp.float32),
        ),
        scratch_types=[
            pltpu.VMEM((_BPW,), jnp.int32),
            pltpu.SemaphoreType.DMA,
            pltpu.SemaphoreType.DMA,
        ],
    )
    def k(u_hbm, mean_hbm, logvar_hbm, mean_out, logvar_out,
          idx_v, sem_m, sem_l):
        wid = lax.axis_index("s") * _NC + lax.axis_index("c")
        base = wid * _BPW
        pltpu.sync_copy(u_hbm.at[pl.ds(base, _BPW)], idx_v)

        @pl.loop(0, _BPW // 16)
        def _(g):
            vr = idx_v[pl.ds(g * 16, 16)]
            for kk in range(16):
                r = vr[kk]
                j = base + g * 16 + kk
                pltpu.async_copy(mean_hbm.at[r], mean_out.at[j], sem_m)
                pltpu.async_copy(logvar_hbm.at[r], logvar_out.at[j], sem_l)

        pltpu.make_async_copy(mean_hbm.at[pl.ds(0, _BPW), :],
                              mean_out.at[pl.ds(base, _BPW)], sem_m).wait()
        pltpu.make_async_copy(logvar_hbm.at[pl.ds(0, _BPW), :],
                              logvar_out.at[pl.ds(base, _BPW)], sem_l).wait()

    return k


_gather2 = jax.jit(_make_kernel())


def kernel(u, mean_table, log_variance_table):
    return _gather2(u, mean_table, log_variance_table)


# speedup vs baseline: 1.8146x; 1.8146x over previous
import functools

import jax
import jax.numpy as jnp
from jax import lax
from jax.experimental import pallas as pl
from jax.experimental.pallas import tpu as pltpu
from jax.experimental.pallas import tpu_sc as plsc

Z = 32
B = 16384

_NC = 2
_NS = 16
_NW = _NC * _NS
_BPW = B // _NW


def _make_kernel():
    mesh = plsc.VectorSubcoreMesh(core_axis_name="c", subcore_axis_name="s")

    @functools.partial(
        pl.kernel,
        mesh=mesh,
        out_type=(
            jax.ShapeDtypeStruct((B, Z), jnp.float32),
            jax.ShapeDtypeStruct((B, Z), jnp.float32),
        ),
        scratch_types=[
            pltpu.VMEM((_BPW,), jnp.int32),
            pltpu.VMEM((_BPW, Z), jnp.float32),
            pltpu.SemaphoreType.DMA,
        ],
    )
    def k(u_hbm, mean_hbm, logvar_hbm, mean_out, logvar_out,
          idx_v, rows_v, sem):
        wid = lax.axis_index("s") * _NC + lax.axis_index("c")
        base = wid * _BPW
        pltpu.sync_copy(u_hbm.at[pl.ds(base, _BPW)], idx_v)

        def gather_one(table_hbm, out_hbm):
            @pl.loop(0, _BPW // 16)
            def _(g):
                vr = idx_v[pl.ds(g * 16, 16)]
                for kk in range(16):
                    r = vr[kk]
                    pltpu.async_copy(table_hbm.at[r],
                                     rows_v.at[g * 16 + kk], sem)

            pltpu.make_async_copy(table_hbm.at[pl.ds(0, _BPW), :],
                                  rows_v, sem).wait()
            pltpu.sync_copy(rows_v, out_hbm.at[pl.ds(base, _BPW)])

        gather_one(mean_hbm, mean_out)
        gather_one(logvar_hbm, logvar_out)

    return k


_gather2 = jax.jit(_make_kernel())


def kernel(u, mean_table, log_variance_table):
    return _gather2(u, mean_table, log_variance_table)
